# baseline (speedup 1.0000x reference)
import functools

import jax
import jax.numpy as jnp
from jax import lax
from jax.experimental import pallas as pl
from jax.experimental.pallas import tpu as pltpu
from jax.experimental.pallas import tpu_sc as plsc

N_ROWS = 2048
N_COLS = 4096
N_THRESH = 32
LANES = 16
N_WORKERS = 32
TC_COLS = 1536
TC_BLOCK = 512
SC_COLS = N_COLS - TC_COLS
COLS_PER_TILE = SC_COLS // N_WORKERS
CHUNK_COLS = LANES
N_CHUNKS = COLS_PER_TILE // CHUNK_COLS
LOG2C = 4
UNROLL = 8
INV_N = 1.0 / N_ROWS


def _sqrt16(v):
    vv = jnp.maximum(v, 1e-30)
    i = lax.bitcast_convert_type(vv, jnp.int32)
    i = 0x5F3759DF - (i >> 1)
    y = lax.bitcast_convert_type(i, jnp.float32)
    for _ in range(3):
        y = y * (1.5 - 0.5 * vv * y * y)
    return vv * y


def _body(x_hbm, t_hbm, st_hbm, out_hbm, xb0, xb1, xb2, tv, t2, tsplat,
          mv, sv, si0, si1, si2, so0, so1, so2):
    wid = lax.axis_index("c") * (N_WORKERS // 2) + lax.axis_index("s")
    col_base = wid * COLS_PER_TILE
    pltpu.sync_copy(t_hbm, tv)
    pltpu.sync_copy(st_hbm.at[0, pl.ds(col_base, COLS_PER_TILE)], mv)
    pltpu.sync_copy(st_hbm.at[1, pl.ds(col_base, COLS_PER_TILE)], sv)
    zi = jnp.zeros((LANES,), jnp.int32)
    lane = lax.iota(jnp.int32, LANES)
    t_lo = tv[pl.ds(0, LANES)]
    t_hi = tv[pl.ds(LANES, LANES)]
    for p in range(N_THRESH):
        src = t_lo if p < LANES else t_hi
        tsplat[p, :] = src[jnp.full((LANES,), p % LANES, jnp.int32)]

    def _build_t2(ck):
        mean = mv[pl.ds(ck * CHUNK_COLS, LANES)]
        stdp = sv[pl.ds(ck * CHUNK_COLS, LANES)]
        for p in range(N_THRESH):
            t2[pl.ds(p * LANES, LANES)] = mean + tsplat[p, :] * stdp

    cb16 = lane + 16 * CHUNK_COLS

    def _ph2(xb, xbo, mean_o, inv_o):
        t15 = t2[pl.ds(15 * LANES, LANES)]
        t7g = t2[pl.ds(7 * LANES, LANES)]
        t23g = t2[pl.ds(23 * LANES, LANES)]
        t31g = t2[pl.ds(31 * LANES, LANES)]

        def bin_one(v):
            m16 = v > t15
            pos = jnp.where(m16, cb16, lane)
            tk8 = jnp.where(m16, t23g, t7g)
            pos = jnp.where(v > tk8, pos + 8 * CHUNK_COLS, pos)
            for k in (4, 2, 1):
                probe = pos + (k - 1) * CHUNK_COLS
                tk = plsc.load_gather(t2, [probe])
                pos = jnp.where(v > tk, probe + CHUNK_COLS, pos)
            pos = jnp.where(v > t31g, pos + CHUNK_COLS, pos)
            return pos >> LOG2C

        def rbody(rb, carry):
            SB, QB = carry
            rbase = rb * UNROLL
            poss = []
            for i in range(UNROLL):
                poss.append(bin_one(xb[rbase + i, :]))
                if xbo is not None:
                    w = xbo[rbase + i, :]
                    xbo[rbase + i, :] = (w - mean_o) * inv_o
            tot = zi
            totq = zi
            for i in range(UNROLL):
                p = poss[i]
                xb[rbase + i, :] = p.astype(jnp.float32)
                tot = tot + p
                totq = totq + p * p
            return (SB + tot, QB + totq)

        SB, QB = lax.fori_loop(0, N_ROWS // UNROLL, rbody, (zi, zi))
        mean = SB.astype(jnp.float32) * INV_N
        var = jnp.maximum(QB.astype(jnp.float32) * INV_N - mean * mean, 0.0)
        inv = 1.0 / (_sqrt16(var) + 1e-6)
        return mean, inv

    def _ph3(xb, mean_o, inv_o):
        def rbody(rb, _r):
            rbase = rb * UNROLL
            for i in range(UNROLL):
                xb[rbase + i, :] = (xb[rbase + i, :] - mean_o) * inv_o
            return 0

        lax.fori_loop(0, N_ROWS // UNROLL, rbody, 0)

    bufs = (xb0, xb1, xb2)
    isems = (si0, si1, si2)
    osems = (so0, so1, so2)

    def _slice(ck):
        c0 = col_base + ck * CHUNK_COLS
        return x_hbm.at[:, pl.ds(TC_COLS + c0, CHUNK_COLS)], \
            out_hbm.at[:, pl.ds(c0, CHUNK_COLS)]

    in_cp = [None, None, None]
    out_cp = [None, None, None]
    src0, _ = _slice(0)
    in_cp[0] = pltpu.async_copy(src0, xb0, si0)
    meanb = invb = None
    for ck in range(N_CHUNKS):
        b = ck % 3
        nb = (ck + 1) % 3
        pb = (ck + 2) % 3
        in_cp[b].wait()
        if ck + 1 < N_CHUNKS:
            if out_cp[nb] is not None:
                out_cp[nb].wait()
            srcn, _ = _slice(ck + 1)
            in_cp[nb] = pltpu.async_copy(srcn, bufs[nb], isems[nb])
        _build_t2(ck)
        if ck == 0:
            meanb, invb = _ph2(bufs[b], None, None, None)
        else:
            meanb, invb = _ph2(bufs[b], bufs[pb], meanb, invb)
            _, dstp = _slice(ck - 1)
            out_cp[pb] = pltpu.async_copy(bufs[pb], dstp, osems[pb])
    bl = (N_CHUNKS - 1) % 3
    _ph3(bufs[bl], meanb, invb)
    _, dstl = _slice(N_CHUNKS - 1)
    out_cp[bl] = pltpu.async_copy(bufs[bl], dstl, osems[bl])
    for cp in out_cp:
        if cp is not None:
            cp.wait()


_sc_call = functools.partial(
    pl.kernel,
    mesh=plsc.VectorSubcoreMesh(core_axis_name="c", subcore_axis_name="s"),
    out_type=jax.ShapeDtypeStruct((N_ROWS, SC_COLS), jnp.float32),
    scratch_types=[
        pltpu.VMEM((N_ROWS, CHUNK_COLS), jnp.float32),
        pltpu.VMEM((N_ROWS, CHUNK_COLS), jnp.float32),
        pltpu.VMEM((N_ROWS, CHUNK_COLS), jnp.float32),
        pltpu.VMEM((N_THRESH,), jnp.float32),
        pltpu.VMEM((N_THRESH * LANES,), jnp.float32),
        pltpu.VMEM((N_THRESH, LANES), jnp.float32),
        pltpu.VMEM((COLS_PER_TILE,), jnp.float32),
        pltpu.VMEM((COLS_PER_TILE,), jnp.float32),
        pltpu.SemaphoreType.DMA,
        pltpu.SemaphoreType.DMA,
        pltpu.SemaphoreType.DMA,
        pltpu.SemaphoreType.DMA,
        pltpu.SemaphoreType.DMA,
        pltpu.SemaphoreType.DMA,
    ],
    compiler_params=pltpu.CompilerParams(
        needs_layout_passes=False, use_tc_tiling_on_sc=False),
)(_body)


def _tc_body(t_ref, x_ref, o_ref):
    xb = x_ref[...]
    mean = jnp.sum(xb, axis=0) * INV_N
    var = jnp.maximum(jnp.sum(xb * xb, axis=0) * INV_N - mean * mean, 0.0)
    stdp = jnp.sqrt(var) + 1e-6
    t = t_ref[...]
    tp_all = mean[None, :] + t[:, None] * stdp[None, :]
    acc = jnp.zeros(xb.shape, jnp.float32)
    for p in range(N_THRESH):
        acc = acc + (xb > tp_all[p, :][None, :]).astype(jnp.float32)
    m2 = jnp.sum(acc, axis=0) * INV_N
    v2 = jnp.maximum(jnp.sum(acc * acc, axis=0) * INV_N - m2 * m2, 0.0)
    inv2 = 1.0 / (jnp.sqrt(v2) + 1e-6)
    o_ref[...] = (acc - m2[None, :]) * inv2[None, :]


_tc_call = pl.pallas_call(
    _tc_body,
    grid=(TC_COLS // TC_BLOCK,),
    in_specs=[
        pl.BlockSpec((N_THRESH,), lambda j: (0,)),
        pl.BlockSpec((N_ROWS, TC_BLOCK), lambda j: (0, j)),
    ],
    out_specs=pl.BlockSpec((N_ROWS, TC_BLOCK), lambda j: (0, j)),
    out_shape=jax.ShapeDtypeStruct((N_ROWS, TC_COLS), jnp.float32),
)


def _stats_body(x_ref, o_ref):
    xb = x_ref[...]
    mean = jnp.sum(xb, axis=0) * INV_N
    var = jnp.maximum(jnp.sum(xb * xb, axis=0) * INV_N - mean * mean, 0.0)
    o_ref[...] = jnp.stack([mean, jnp.sqrt(var) + 1e-6], axis=0)


_stats_call = pl.pallas_call(
    _stats_body,
    grid=(SC_COLS // TC_BLOCK,),
    in_specs=[
        pl.BlockSpec((N_ROWS, TC_BLOCK),
                     lambda j: (0, j + TC_COLS // TC_BLOCK)),
    ],
    out_specs=pl.BlockSpec((2, TC_BLOCK), lambda j: (0, j)),
    out_shape=jax.ShapeDtypeStruct((2, SC_COLS), jnp.float32),
)


def kernel(x, thresholds):
    ts = jnp.sort(thresholds)
    stats = _stats_call(x)
    out_sc = _sc_call(x, ts, stats)
    out_tc = _tc_call(ts, x)
    return jnp.concatenate([out_tc, out_sc], axis=1)


# speedup vs baseline: 1.4901x; 1.4901x over previous
import functools

import jax
import jax.numpy as jnp
from jax import lax
from jax.experimental import pallas as pl
from jax.experimental.pallas import tpu as pltpu
from jax.experimental.pallas import tpu_sc as plsc

N_ROWS = 2048
N_COLS = 4096
N_THRESH = 32
LANES = 16
N_WORKERS = 32
TC_COLS = 1536
TC_BLOCK = 256
SC_COLS = N_COLS - TC_COLS
COLS_PER_TILE = SC_COLS // N_WORKERS
CHUNK_COLS = LANES
N_CHUNKS = COLS_PER_TILE // CHUNK_COLS
LOG2C = 4
UNROLL = 8
INV_N = 1.0 / N_ROWS


def _sqrt16(v):
    vv = jnp.maximum(v, 1e-30)
    i = lax.bitcast_convert_type(vv, jnp.int32)
    i = 0x5F3759DF - (i >> 1)
    y = lax.bitcast_convert_type(i, jnp.float32)
    for _ in range(3):
        y = y * (1.5 - 0.5 * vv * y * y)
    return vv * y


def _body(x_hbm, t_hbm, out_hbm, xb0, xb1, xb2, tv, t2, tsplat,
          si0, si1, si2, so0, so1, so2):
    wid = lax.axis_index("c") * (N_WORKERS // 2) + lax.axis_index("s")
    col_base = wid * COLS_PER_TILE
    pltpu.sync_copy(t_hbm, tv)
    zf = jnp.zeros((LANES,), jnp.float32)
    zi = jnp.zeros((LANES,), jnp.int32)
    lane = lax.iota(jnp.int32, LANES)
    t_lo = tv[pl.ds(0, LANES)]
    t_hi = tv[pl.ds(LANES, LANES)]
    for p in range(N_THRESH):
        src = t_lo if p < LANES else t_hi
        tsplat[p, :] = src[jnp.full((LANES,), p % LANES, jnp.int32)]

    def _ph1(xb, xbo, mean_o, inv_o):
        def rbody(rb, carry):
            Ss, Qs = carry
            rbase = rb * UNROLL
            Ss2, Qs2 = [], []
            for i in range(UNROLL):
                v = xb[rbase + i, :]
                Ss2.append(Ss[i] + v)
                Qs2.append(Qs[i] + v * v)
                if xbo is not None:
                    w = xbo[rbase + i, :]
                    xbo[rbase + i, :] = (w - mean_o) * inv_o
            return (tuple(Ss2), tuple(Qs2))

        Ss, Qs = lax.fori_loop(0, N_ROWS // UNROLL, rbody,
                               ((zf,) * UNROLL, (zf,) * UNROLL))
        S, Q = zf, zf
        for i in range(UNROLL):
            S = S + Ss[i]
            Q = Q + Qs[i]
        return S, Q

    def _build_t2(S, Q):
        mean = S * INV_N
        var = jnp.maximum(Q * INV_N - mean * mean, 0.0)
        stdp = _sqrt16(var) + 1e-6
        for p in range(N_THRESH):
            t2[pl.ds(p * LANES, LANES)] = mean + tsplat[p, :] * stdp

    cb16 = lane + 16 * CHUNK_COLS

    def _ph2(xb):
        t15 = t2[pl.ds(15 * LANES, LANES)]
        t7g = t2[pl.ds(7 * LANES, LANES)]
        t23g = t2[pl.ds(23 * LANES, LANES)]
        t31g = t2[pl.ds(31 * LANES, LANES)]

        def bin_one(v):
            m16 = v > t15
            pos = jnp.where(m16, cb16, lane)
            tk8 = jnp.where(m16, t23g, t7g)
            pos = jnp.where(v > tk8, pos + 8 * CHUNK_COLS, pos)
            for k in (4, 2, 1):
                probe = pos + (k - 1) * CHUNK_COLS
                tk = plsc.load_gather(t2, [probe])
                pos = jnp.where(v > tk, probe + CHUNK_COLS, pos)
            pos = jnp.where(v > t31g, pos + CHUNK_COLS, pos)
            return pos >> LOG2C

        def rbody(rb, carry):
            SB, QB = carry
            rbase = rb * UNROLL
            poss = []
            for i in range(UNROLL):
                poss.append(bin_one(xb[rbase + i, :]))
            tot = zi
            totq = zi
            for i in range(UNROLL):
                p = poss[i]
                xb[rbase + i, :] = p.astype(jnp.float32)
                tot = tot + p
                totq = totq + p * p
            return (SB + tot, QB + totq)

        SB, QB = lax.fori_loop(0, N_ROWS // UNROLL, rbody, (zi, zi))
        mean = SB.astype(jnp.float32) * INV_N
        var = jnp.maximum(QB.astype(jnp.float32) * INV_N - mean * mean, 0.0)
        inv = 1.0 / (_sqrt16(var) + 1e-6)
        return mean, inv

    def _ph3(xb, mean_o, inv_o):
        def rbody(rb, _r):
            rbase = rb * UNROLL
            for i in range(UNROLL):
                xb[rbase + i, :] = (xb[rbase + i, :] - mean_o) * inv_o
            return 0

        lax.fori_loop(0, N_ROWS // UNROLL, rbody, 0)

    bufs = (xb0, xb1, xb2)
    isems = (si0, si1, si2)
    osems = (so0, so1, so2)

    def _slice(ck):
        c0 = col_base + ck * CHUNK_COLS
        return x_hbm.at[:, pl.ds(TC_COLS + c0, CHUNK_COLS)], \
            out_hbm.at[:, pl.ds(c0, CHUNK_COLS)]

    in_cp = [None, None, None]
    out_cp = [None, None, None]
    src0, _ = _slice(0)
    in_cp[0] = pltpu.async_copy(src0, xb0, si0)
    meanb = invb = None
    for ck in range(N_CHUNKS):
        b = ck % 3
        nb = (ck + 1) % 3
        pb = (ck + 2) % 3
        in_cp[b].wait()
        if ck + 1 < N_CHUNKS:
            if out_cp[nb] is not None:
                out_cp[nb].wait()
            srcn, _ = _slice(ck + 1)
            in_cp[nb] = pltpu.async_copy(srcn, bufs[nb], isems[nb])
        if ck == 0:
            S, Q = _ph1(bufs[b], None, None, None)
        else:
            S, Q = _ph1(bufs[b], bufs[pb], meanb, invb)
            _, dstp = _slice(ck - 1)
            out_cp[pb] = pltpu.async_copy(bufs[pb], dstp, osems[pb])
        _build_t2(S, Q)
        meanb, invb = _ph2(bufs[b])
    bl = (N_CHUNKS - 1) % 3
    _ph3(bufs[bl], meanb, invb)
    _, dstl = _slice(N_CHUNKS - 1)
    out_cp[bl] = pltpu.async_copy(bufs[bl], dstl, osems[bl])
    for cp in out_cp:
        if cp is not None:
            cp.wait()


_sc_call = functools.partial(
    pl.kernel,
    mesh=plsc.VectorSubcoreMesh(core_axis_name="c", subcore_axis_name="s"),
    out_type=jax.ShapeDtypeStruct((N_ROWS, SC_COLS), jnp.float32),
    scratch_types=[
        pltpu.VMEM((N_ROWS, CHUNK_COLS), jnp.float32),
        pltpu.VMEM((N_ROWS, CHUNK_COLS), jnp.float32),
        pltpu.VMEM((N_ROWS, CHUNK_COLS), jnp.float32),
        pltpu.VMEM((N_THRESH,), jnp.float32),
        pltpu.VMEM((N_THRESH * LANES,), jnp.float32),
        pltpu.VMEM((N_THRESH, LANES), jnp.float32),
        pltpu.SemaphoreType.DMA,
        pltpu.SemaphoreType.DMA,
        pltpu.SemaphoreType.DMA,
        pltpu.SemaphoreType.DMA,
        pltpu.SemaphoreType.DMA,
        pltpu.SemaphoreType.DMA,
    ],
    compiler_params=pltpu.CompilerParams(
        needs_layout_passes=False, use_tc_tiling_on_sc=False),
)(_body)


def _tc_body(t_ref, x_ref, o_ref):
    xb = x_ref[...]
    mean = jnp.sum(xb, axis=0) * INV_N
    var = jnp.maximum(jnp.sum(xb * xb, axis=0) * INV_N - mean * mean, 0.0)
    stdp = jnp.sqrt(var) + 1e-6
    t = t_ref[...]
    tp_all = mean[None, :] + t[:, None] * stdp[None, :]
    acc = jnp.zeros(xb.shape, jnp.float32)
    for p in range(N_THRESH):
        acc = acc + (xb > tp_all[p, :][None, :]).astype(jnp.float32)
    m2 = jnp.sum(acc, axis=0) * INV_N
    v2 = jnp.maximum(jnp.sum(acc * acc, axis=0) * INV_N - m2 * m2, 0.0)
    inv2 = 1.0 / (jnp.sqrt(v2) + 1e-6)
    o_ref[...] = (acc - m2[None, :]) * inv2[None, :]


_tc_call = pl.pallas_call(
    _tc_body,
    grid=(TC_COLS // TC_BLOCK,),
    in_specs=[
        pl.BlockSpec((N_THRESH,), lambda j: (0,)),
        pl.BlockSpec((N_ROWS, TC_BLOCK), lambda j: (0, j)),
    ],
    out_specs=pl.BlockSpec((N_ROWS, TC_BLOCK), lambda j: (0, j)),
    out_shape=jax.ShapeDtypeStruct((N_ROWS, TC_COLS), jnp.float32),
)


def kernel(x, thresholds):
    ts = jnp.sort(thresholds)
    out_sc = _sc_call(x, ts)
    out_tc = _tc_call(ts, x)
    return jnp.concatenate([out_tc, out_sc], axis=1)


# speedup vs baseline: 1.5188x; 1.0192x over previous
import functools

import jax
import jax.numpy as jnp
from jax import lax
from jax.experimental import pallas as pl
from jax.experimental.pallas import tpu as pltpu
from jax.experimental.pallas import tpu_sc as plsc

N_ROWS = 2048
N_COLS = 4096
N_THRESH = 32
LANES = 16
N_WORKERS = 32
TC_COLS = 1536
TC_BLOCK = 256
SC_COLS = N_COLS - TC_COLS
COLS_PER_TILE = SC_COLS // N_WORKERS
CHUNK_COLS = LANES
N_CHUNKS = COLS_PER_TILE // CHUNK_COLS
LOG2C = 4
UNROLL = 8
INV_N = 1.0 / N_ROWS


def _sqrt16(v):
    vv = jnp.maximum(v, 1e-30)
    i = lax.bitcast_convert_type(vv, jnp.int32)
    i = 0x5F3759DF - (i >> 1)
    y = lax.bitcast_convert_type(i, jnp.float32)
    for _ in range(3):
        y = y * (1.5 - 0.5 * vv * y * y)
    return vv * y


def _body(x_hbm, t_hbm, out_hbm, xb0, xb1, xb2, tv, t2, tsplat,
          si0, si1, si2, so0, so1, so2):
    wid = lax.axis_index("c") * (N_WORKERS // 2) + lax.axis_index("s")
    col_base = wid * COLS_PER_TILE
    pltpu.sync_copy(t_hbm, tv)
    zf = jnp.zeros((LANES,), jnp.float32)
    zi = jnp.zeros((LANES,), jnp.int32)
    lane = lax.iota(jnp.int32, LANES)
    t_lo = tv[pl.ds(0, LANES)]
    t_hi = tv[pl.ds(LANES, LANES)]
    for p in range(N_THRESH):
        src = t_lo if p < LANES else t_hi
        tsplat[p, :] = src[jnp.full((LANES,), p % LANES, jnp.int32)]

    def _ph1(xb, xbo, inv_o, neg_mi_o):
        def rbody(rb, carry):
            Ss, Qs = carry
            rbase = rb * UNROLL
            Ss2, Qs2 = [], []
            for i in range(UNROLL):
                v = xb[rbase + i, :]
                Ss2.append(Ss[i] + v)
                Qs2.append(Qs[i] + v * v)
                if xbo is not None:
                    w = xbo[rbase + i, :]
                    xbo[rbase + i, :] = w * inv_o + neg_mi_o
            return (tuple(Ss2), tuple(Qs2))

        Ss, Qs = lax.fori_loop(0, N_ROWS // UNROLL, rbody,
                               ((zf,) * UNROLL, (zf,) * UNROLL))
        S, Q = zf, zf
        for i in range(UNROLL):
            S = S + Ss[i]
            Q = Q + Qs[i]
        return S, Q

    def _build_t2(S, Q):
        mean = S * INV_N
        var = jnp.maximum(Q * INV_N - mean * mean, 0.0)
        stdp = _sqrt16(var) + 1e-6
        for p in range(N_THRESH):
            t2[pl.ds(p * LANES, LANES)] = mean + tsplat[p, :] * stdp

    cb16 = lane + 16 * CHUNK_COLS

    def _ph2(xb):
        t15 = t2[pl.ds(15 * LANES, LANES)]
        t7g = t2[pl.ds(7 * LANES, LANES)]
        t23g = t2[pl.ds(23 * LANES, LANES)]
        t31g = t2[pl.ds(31 * LANES, LANES)]

        def bin_one(v):
            m16 = v > t15
            pos = jnp.where(m16, cb16, lane)
            tk8 = jnp.where(m16, t23g, t7g)
            pos = jnp.where(v > tk8, pos + 8 * CHUNK_COLS, pos)
            for k in (4, 2, 1):
                probe = pos + (k - 1) * CHUNK_COLS
                tk = plsc.load_gather(t2, [probe])
                pos = jnp.where(v > tk, probe + CHUNK_COLS, pos)
            pos = jnp.where(v > t31g, pos + CHUNK_COLS, pos)
            return pos >> LOG2C

        def rbody(rb, carry):
            SB, QB = carry
            rbase = rb * UNROLL
            poss = []
            for i in range(UNROLL):
                poss.append(bin_one(xb[rbase + i, :]))
            tot = zf
            totq = zf
            for i in range(UNROLL):
                pf = poss[i].astype(jnp.float32)
                xb[rbase + i, :] = pf
                tot = tot + pf
                totq = totq + pf * pf
            return (SB + tot, QB + totq)

        SB, QB = lax.fori_loop(0, N_ROWS // UNROLL, rbody, (zf, zf))
        mean = SB * INV_N
        var = jnp.maximum(QB * INV_N - mean * mean, 0.0)
        inv = 1.0 / (_sqrt16(var) + 1e-6)
        return inv, -mean * inv

    def _ph3(xb, inv_o, neg_mi_o):
        def rbody(rb, _r):
            rbase = rb * UNROLL
            for i in range(UNROLL):
                xb[rbase + i, :] = xb[rbase + i, :] * inv_o + neg_mi_o
            return 0

        lax.fori_loop(0, N_ROWS // UNROLL, rbody, 0)

    bufs = (xb0, xb1, xb2)
    isems = (si0, si1, si2)
    osems = (so0, so1, so2)

    def _slice(ck):
        c0 = col_base + ck * CHUNK_COLS
        return x_hbm.at[:, pl.ds(TC_COLS + c0, CHUNK_COLS)], \
            out_hbm.at[:, pl.ds(c0, CHUNK_COLS)]

    in_cp = [None, None, None]
    out_cp = [None, None, None]
    src0, _ = _slice(0)
    in_cp[0] = pltpu.async_copy(src0, xb0, si0)
    invb = nmib = None
    for ck in range(N_CHUNKS):
        b = ck % 3
        nb = (ck + 1) % 3
        pb = (ck + 2) % 3
        in_cp[b].wait()
        if ck + 1 < N_CHUNKS:
            if out_cp[nb] is not None:
                out_cp[nb].wait()
            srcn, _ = _slice(ck + 1)
            in_cp[nb] = pltpu.async_copy(srcn, bufs[nb], isems[nb])
        if ck == 0:
            S, Q = _ph1(bufs[b], None, None, None)
        else:
            S, Q = _ph1(bufs[b], bufs[pb], invb, nmib)
            _, dstp = _slice(ck - 1)
            out_cp[pb] = pltpu.async_copy(bufs[pb], dstp, osems[pb])
        _build_t2(S, Q)
        invb, nmib = _ph2(bufs[b])
    bl = (N_CHUNKS - 1) % 3
    _ph3(bufs[bl], invb, nmib)
    _, dstl = _slice(N_CHUNKS - 1)
    out_cp[bl] = pltpu.async_copy(bufs[bl], dstl, osems[bl])
    for cp in out_cp:
        if cp is not None:
            cp.wait()


_sc_call = functools.partial(
    pl.kernel,
    mesh=plsc.VectorSubcoreMesh(core_axis_name="c", subcore_axis_name="s"),
    out_type=jax.ShapeDtypeStruct((N_ROWS, SC_COLS), jnp.float32),
    scratch_types=[
        pltpu.VMEM((N_ROWS, CHUNK_COLS), jnp.float32),
        pltpu.VMEM((N_ROWS, CHUNK_COLS), jnp.float32),
        pltpu.VMEM((N_ROWS, CHUNK_COLS), jnp.float32),
        pltpu.VMEM((N_THRESH,), jnp.float32),
        pltpu.VMEM((N_THRESH * LANES,), jnp.float32),
        pltpu.VMEM((N_THRESH, LANES), jnp.float32),
        pltpu.SemaphoreType.DMA,
        pltpu.SemaphoreType.DMA,
        pltpu.SemaphoreType.DMA,
        pltpu.SemaphoreType.DMA,
        pltpu.SemaphoreType.DMA,
        pltpu.SemaphoreType.DMA,
    ],
    compiler_params=pltpu.CompilerParams(
        needs_layout_passes=False, use_tc_tiling_on_sc=False),
)(_body)


def _tc_body(t_ref, x_ref, o_ref):
    xb = x_ref[...]
    mean = jnp.sum(xb, axis=0) * INV_N
    var = jnp.maximum(jnp.sum(xb * xb, axis=0) * INV_N - mean * mean, 0.0)
    stdp = jnp.sqrt(var) + 1e-6
    t = t_ref[...]
    tp_all = mean[None, :] + t[:, None] * stdp[None, :]
    acc = jnp.zeros(xb.shape, jnp.float32)
    for p in range(N_THRESH):
        acc = acc + (xb > tp_all[p, :][None, :]).astype(jnp.float32)
    m2 = jnp.sum(acc, axis=0) * INV_N
    v2 = jnp.maximum(jnp.sum(acc * acc, axis=0) * INV_N - m2 * m2, 0.0)
    inv2 = 1.0 / (jnp.sqrt(v2) + 1e-6)
    o_ref[...] = (acc - m2[None, :]) * inv2[None, :]


_tc_call = pl.pallas_call(
    _tc_body,
    grid=(TC_COLS // TC_BLOCK,),
    in_specs=[
        pl.BlockSpec((N_THRESH,), lambda j: (0,)),
        pl.BlockSpec((N_ROWS, TC_BLOCK), lambda j: (0, j)),
    ],
    out_specs=pl.BlockSpec((N_ROWS, TC_BLOCK), lambda j: (0, j)),
    out_shape=jax.ShapeDtypeStruct((N_ROWS, TC_COLS), jnp.float32),
)


def kernel(x, thresholds):
    ts = jnp.sort(thresholds)
    out_sc = _sc_call(x, ts)
    out_tc = _tc_call(ts, x)
    return jnp.concatenate([out_tc, out_sc], axis=1)


# speedup vs baseline: 1.5613x; 1.0280x over previous
import functools

import jax
import jax.numpy as jnp
from jax import lax
from jax.experimental import pallas as pl
from jax.experimental.pallas import tpu as pltpu
from jax.experimental.pallas import tpu_sc as plsc

N_ROWS = 2048
N_COLS = 4096
N_THRESH = 32
LANES = 16
N_WORKERS = 32
TC_COLS = 1536
TC_BLOCK = 256
SC_COLS = N_COLS - TC_COLS
COLS_PER_TILE = SC_COLS // N_WORKERS
CHUNK_COLS = LANES
N_CHUNKS = COLS_PER_TILE // CHUNK_COLS
LOG2C = 4
UNROLL = 8
INV_N = 1.0 / N_ROWS


def _sqrt16(v):
    vv = jnp.maximum(v, 1e-30)
    i = lax.bitcast_convert_type(vv, jnp.int32)
    i = 0x5F3759DF - (i >> 1)
    y = lax.bitcast_convert_type(i, jnp.float32)
    for _ in range(3):
        y = y * (1.5 - 0.5 * vv * y * y)
    return vv * y


def _body(x_hbm, t_hbm, out_hbm, xb0, xb1, xb2, tv, t2, tsplat,
          si0, si1, si2, so0, so1, so2):
    wid = lax.axis_index("c") * (N_WORKERS // 2) + lax.axis_index("s")
    col_base = wid * COLS_PER_TILE
    pltpu.sync_copy(t_hbm, tv)
    zf = jnp.zeros((LANES,), jnp.float32)
    zi = jnp.zeros((LANES,), jnp.int32)
    lane = lax.iota(jnp.int32, LANES)
    t_lo = tv[pl.ds(0, LANES)]
    t_hi = tv[pl.ds(LANES, LANES)]
    for p in range(N_THRESH):
        src = t_lo if p < LANES else t_hi
        tsplat[p, :] = src[jnp.full((LANES,), p % LANES, jnp.int32)]

    def _ph1(xb, xbo, inv_o, neg_mi_o):
        def rbody(rb, carry):
            Ss, Qs = carry
            rbase = rb * UNROLL
            Ss2, Qs2 = [], []
            for i in range(UNROLL):
                v = xb[rbase + i, :]
                Ss2.append(Ss[i] + v)
                Qs2.append(Qs[i] + v * v)
                if xbo is not None:
                    w = xbo[rbase + i, :]
                    xbo[rbase + i, :] = w * inv_o + neg_mi_o
            return (tuple(Ss2), tuple(Qs2))

        Ss, Qs = lax.fori_loop(0, N_ROWS // UNROLL, rbody,
                               ((zf,) * UNROLL, (zf,) * UNROLL))
        S, Q = zf, zf
        for i in range(UNROLL):
            S = S + Ss[i]
            Q = Q + Qs[i]
        return S, Q

    def _build_t2(S, Q):
        mean = S * INV_N
        var = jnp.maximum(Q * INV_N - mean * mean, 0.0)
        stdp = _sqrt16(var) + 1e-6
        for p in range(N_THRESH):
            t2[pl.ds(p * LANES, LANES)] = mean + tsplat[p, :] * stdp

    cb16 = lane + 16 * CHUNK_COLS

    def _ph2(xb):
        t15 = t2[pl.ds(15 * LANES, LANES)]
        t7g = t2[pl.ds(7 * LANES, LANES)]
        t23g = t2[pl.ds(23 * LANES, LANES)]
        t31g = t2[pl.ds(31 * LANES, LANES)]

        def bin_one(v):
            m16 = v > t15
            pos = jnp.where(m16, cb16, lane)
            tk8 = jnp.where(m16, t23g, t7g)
            pos = jnp.where(v > tk8, pos + 8 * CHUNK_COLS, pos)
            for k in (4, 2, 1):
                off = (k - 1) * CHUNK_COLS
                tk = plsc.load_gather(
                    t2.at[pl.ds(off, N_THRESH * LANES - off)], [pos])
                pos = jnp.where(v > tk, pos + (off + CHUNK_COLS), pos)
            pos = jnp.where(v > t31g, pos + CHUNK_COLS, pos)
            return pos >> LOG2C

        def rbody(rb, carry):
            SB, QB = carry
            rbase = rb * UNROLL
            poss = []
            for i in range(UNROLL):
                poss.append(bin_one(xb[rbase + i, :]))
            tot = zf
            totq = zf
            for i in range(UNROLL):
                pf = poss[i].astype(jnp.float32)
                xb[rbase + i, :] = pf
                tot = tot + pf
                totq = totq + pf * pf
            return (SB + tot, QB + totq)

        SB, QB = lax.fori_loop(0, N_ROWS // UNROLL, rbody, (zf, zf))
        mean = SB * INV_N
        var = jnp.maximum(QB * INV_N - mean * mean, 0.0)
        inv = 1.0 / (_sqrt16(var) + 1e-6)
        return inv, -mean * inv

    def _ph3(xb, inv_o, neg_mi_o):
        def rbody(rb, _r):
            rbase = rb * UNROLL
            for i in range(UNROLL):
                xb[rbase + i, :] = xb[rbase + i, :] * inv_o + neg_mi_o
            return 0

        lax.fori_loop(0, N_ROWS // UNROLL, rbody, 0)

    bufs = (xb0, xb1, xb2)
    isems = (si0, si1, si2)
    osems = (so0, so1, so2)

    def _slice(ck):
        c0 = col_base + ck * CHUNK_COLS
        return x_hbm.at[:, pl.ds(TC_COLS + c0, CHUNK_COLS)], \
            out_hbm.at[:, pl.ds(c0, CHUNK_COLS)]

    in_cp = [None, None, None]
    out_cp = [None, None, None]
    src0, _ = _slice(0)
    in_cp[0] = pltpu.async_copy(src0, xb0, si0)
    invb = nmib = None
    for ck in range(N_CHUNKS):
        b = ck % 3
        nb = (ck + 1) % 3
        pb = (ck + 2) % 3
        in_cp[b].wait()
        if ck + 1 < N_CHUNKS:
            if out_cp[nb] is not None:
                out_cp[nb].wait()
            srcn, _ = _slice(ck + 1)
            in_cp[nb] = pltpu.async_copy(srcn, bufs[nb], isems[nb])
        if ck == 0:
            S, Q = _ph1(bufs[b], None, None, None)
        else:
            S, Q = _ph1(bufs[b], bufs[pb], invb, nmib)
            _, dstp = _slice(ck - 1)
            out_cp[pb] = pltpu.async_copy(bufs[pb], dstp, osems[pb])
        _build_t2(S, Q)
        invb, nmib = _ph2(bufs[b])
    bl = (N_CHUNKS - 1) % 3
    _ph3(bufs[bl], invb, nmib)
    _, dstl = _slice(N_CHUNKS - 1)
    out_cp[bl] = pltpu.async_copy(bufs[bl], dstl, osems[bl])
    for cp in out_cp:
        if cp is not None:
            cp.wait()


_sc_call = functools.partial(
    pl.kernel,
    mesh=plsc.VectorSubcoreMesh(core_axis_name="c", subcore_axis_name="s"),
    out_type=jax.ShapeDtypeStruct((N_ROWS, SC_COLS), jnp.float32),
    scratch_types=[
        pltpu.VMEM((N_ROWS, CHUNK_COLS), jnp.float32),
        pltpu.VMEM((N_ROWS, CHUNK_COLS), jnp.float32),
        pltpu.VMEM((N_ROWS, CHUNK_COLS), jnp.float32),
        pltpu.VMEM((N_THRESH,), jnp.float32),
        pltpu.VMEM((N_THRESH * LANES,), jnp.float32),
        pltpu.VMEM((N_THRESH, LANES), jnp.float32),
        pltpu.SemaphoreType.DMA,
        pltpu.SemaphoreType.DMA,
        pltpu.SemaphoreType.DMA,
        pltpu.SemaphoreType.DMA,
        pltpu.SemaphoreType.DMA,
        pltpu.SemaphoreType.DMA,
    ],
    compiler_params=pltpu.CompilerParams(
        needs_layout_passes=False, use_tc_tiling_on_sc=False),
)(_body)


def _tc_body(t_ref, x_ref, o_ref):
    xb = x_ref[...]
    mean = jnp.sum(xb, axis=0) * INV_N
    var = jnp.maximum(jnp.sum(xb * xb, axis=0) * INV_N - mean * mean, 0.0)
    stdp = jnp.sqrt(var) + 1e-6
    t = t_ref[...]
    tp_all = mean[None, :] + t[:, None] * stdp[None, :]
    acc = jnp.zeros(xb.shape, jnp.float32)
    for p in range(N_THRESH):
        acc = acc + (xb > tp_all[p, :][None, :]).astype(jnp.float32)
    m2 = jnp.sum(acc, axis=0) * INV_N
    v2 = jnp.maximum(jnp.sum(acc * acc, axis=0) * INV_N - m2 * m2, 0.0)
    inv2 = 1.0 / (jnp.sqrt(v2) + 1e-6)
    o_ref[...] = (acc - m2[None, :]) * inv2[None, :]


_tc_call = pl.pallas_call(
    _tc_body,
    grid=(TC_COLS // TC_BLOCK,),
    in_specs=[
        pl.BlockSpec((N_THRESH,), lambda j: (0,)),
        pl.BlockSpec((N_ROWS, TC_BLOCK), lambda j: (0, j)),
    ],
    out_specs=pl.BlockSpec((N_ROWS, TC_BLOCK), lambda j: (0, j)),
    out_shape=jax.ShapeDtypeStruct((N_ROWS, TC_COLS), jnp.float32),
)


def kernel(x, thresholds):
    ts = jnp.sort(thresholds)
    out_sc = _sc_call(x, ts)
    out_tc = _tc_call(ts, x)
    return jnp.concatenate([out_tc, out_sc], axis=1)
